# baseline (speedup 1.0000x reference)
import jax
import jax.numpy as jnp
from jax.experimental import pallas as pl
from jax.experimental.pallas import tpu as pltpu

H, W, C = 512, 512, 3
S = 8
QD, KD = 32, 32
TOPK = 64
NP = (H // S) * (W // S)
PDIM = S * S * C
RSC = S * C
NPI = H // S
NPJ = W // S
RB = 512
NBLK = NP // RB
LOG2E = 1.4426950408889634


def _fused_kernel(inp_ref, wq_ref, bq_ref, wk_ref, bk_ref,
                  colsum_ref, bests_ref, idx_ref, q_scr, k_scr):
    i = pl.program_id(0)

    @pl.when(i == 0)
    def _init():
        colsum_ref[...] = jnp.zeros_like(colsum_ref)
        q_scr[...] = jnp.broadcast_to(bq_ref[...] * LOG2E, (NP, QD))
        k_scr[...] = jnp.broadcast_to(bk_ref[...], (NP, KD))

    @pl.when(i < S)
    def _project():
        x = inp_ref[:, 0, :, :].reshape(NP, RSC)
        q_scr[...] += jnp.dot(x, wq_ref[0] * LOG2E,
                              preferred_element_type=jnp.float32)
        k_scr[...] += jnp.dot(x, wk_ref[0],
                              preferred_element_type=jnp.float32)

    @pl.when(i >= S)
    def _attend():
        j = i - S
        q_blk = q_scr[pl.ds(j * RB, RB), :]
        s = jax.lax.dot_general(
            q_blk, k_scr[...], (((1,), (1,)), ((), ())),
            preferred_element_type=jnp.float32)
        m = jnp.max(s, axis=1, keepdims=True)
        p = jnp.exp2(s - m)
        z = jnp.dot(p, jnp.ones((NP, 1), jnp.float32),
                    preferred_element_type=jnp.float32)
        invz = 1.0 / z
        colsum_ref[...] += jax.lax.dot_general(
            invz, p, (((0,), (0,)), ((), ())),
            preferred_element_type=jnp.float32)

    @pl.when(i == S + NBLK - 1)
    def _topk():
        lanes = jax.lax.broadcasted_iota(jnp.int32, (1, NP), 1)
        tlanes = jax.lax.broadcasted_iota(jnp.int32, (1, TOPK), 1)

        def body(t, carry):
            cur, bvals, bidx = carry
            mval = jnp.max(cur)
            midx = jnp.min(jnp.where(cur == mval, lanes, NP))
            bvals = jnp.where(tlanes == t, mval, bvals)
            bidx = jnp.where(tlanes == t, midx, bidx)
            cur = jnp.where(lanes == midx, -jnp.inf, cur)
            return cur, bvals, bidx

        _, bvals, bidx = jax.lax.fori_loop(
            0, TOPK, body,
            (colsum_ref[...],
             jnp.zeros((1, TOPK), jnp.float32),
             jnp.zeros((1, TOPK), jnp.int32)))
        bests_ref[...] = bvals
        idx_ref[...] = bidx


def kernel(input, Wq, bq, Wk, bk):
    inp4 = input.reshape(NPI, S, NPJ, RSC)
    wq4 = Wq.reshape(S, RSC, QD)
    wk4 = Wk.reshape(S, RSC, KD)
    colsum, bests, idx = pl.pallas_call(
        _fused_kernel,
        grid=(S + NBLK,),
        in_specs=[
            pl.BlockSpec((NPI, 1, NPJ, RSC),
                         lambda i: (0, jnp.minimum(i, S - 1), 0, 0)),
            pl.BlockSpec((1, RSC, QD), lambda i: (jnp.minimum(i, S - 1), 0, 0)),
            pl.BlockSpec((1, QD), lambda i: (0, 0)),
            pl.BlockSpec((1, RSC, KD), lambda i: (jnp.minimum(i, S - 1), 0, 0)),
            pl.BlockSpec((1, KD), lambda i: (0, 0)),
        ],
        out_specs=[
            pl.BlockSpec((1, NP), lambda i: (0, 0)),
            pl.BlockSpec((1, TOPK), lambda i: (0, 0)),
            pl.BlockSpec((1, TOPK), lambda i: (0, 0)),
        ],
        out_shape=[
            jax.ShapeDtypeStruct((1, NP), jnp.float32),
            jax.ShapeDtypeStruct((1, TOPK), jnp.float32),
            jax.ShapeDtypeStruct((1, TOPK), jnp.int32),
        ],
        scratch_shapes=[
            pltpu.VMEM((NP, QD), jnp.float32),
            pltpu.VMEM((NP, KD), jnp.float32),
        ],
    )(inp4, wq4, bq.reshape(1, QD), wk4, bk.reshape(1, KD))
    return bests[0], idx[0], colsum[0]


# speedup vs baseline: 1.1389x; 1.1389x over previous
import jax
import jax.numpy as jnp
from jax.experimental import pallas as pl
from jax.experimental.pallas import tpu as pltpu

H, W, C = 512, 512, 3
S = 8
QD, KD = 32, 32
TOPK = 64
NP = (H // S) * (W // S)
PDIM = S * S * C
RB = 512
NBLK = NP // RB
LOG2E = 1.4426950408889634


def _fused_kernel(patches_ref, wq_ref, bq_ref, wk_ref, bk_ref,
                  colsum_ref, bests_ref, idx_ref, k_scratch):
    i = pl.program_id(0)

    @pl.when(i == 0)
    def _init():
        k_scratch[...] = (
            jnp.dot(patches_ref[...], wk_ref[...],
                    preferred_element_type=jnp.float32) + bk_ref[...])
        colsum_ref[...] = jnp.zeros_like(colsum_ref)

    p_blk = patches_ref[pl.ds(i * RB, RB), :]
    q = (jnp.dot(p_blk, wq_ref[...], preferred_element_type=jnp.float32)
         + bq_ref[...])
    s = jax.lax.dot_general(
        q, k_scratch[...], (((1,), (1,)), ((), ())),
        preferred_element_type=jnp.float32)
    m = jnp.max(s, axis=1, keepdims=True)
    p = jnp.exp2(s - m)
    z = jnp.sum(p, axis=1, keepdims=True)
    colsum_ref[...] += jnp.sum(p / z, axis=0, keepdims=True)

    @pl.when(i == NBLK - 1)
    def _topk():
        lanes = jax.lax.broadcasted_iota(jnp.int32, (1, NP), 1)
        tlanes = jax.lax.broadcasted_iota(jnp.int32, (1, TOPK), 1)

        def body(t, carry):
            cur, bvals, bidx = carry
            mval = jnp.max(cur)
            midx = jnp.min(jnp.where(cur == mval, lanes, NP))
            bvals = jnp.where(tlanes == t, mval, bvals)
            bidx = jnp.where(tlanes == t, midx, bidx)
            cur = jnp.where(lanes == midx, -jnp.inf, cur)
            return cur, bvals, bidx

        _, bvals, bidx = jax.lax.fori_loop(
            0, TOPK, body,
            (colsum_ref[...],
             jnp.zeros((1, TOPK), jnp.float32),
             jnp.zeros((1, TOPK), jnp.int32)))
        bests_ref[...] = bvals
        idx_ref[...] = bidx


def kernel(input, Wq, bq, Wk, bk):
    patches = input.reshape(H // S, S, W // S, S * C)
    patches = patches.transpose(0, 2, 1, 3).reshape(NP, PDIM)
    colsum, bests, idx = pl.pallas_call(
        _fused_kernel,
        grid=(NBLK,),
        in_specs=[
            pl.BlockSpec((NP, PDIM), lambda i: (0, 0)),
            pl.BlockSpec((PDIM, QD), lambda i: (0, 0)),
            pl.BlockSpec((1, QD), lambda i: (0, 0)),
            pl.BlockSpec((PDIM, KD), lambda i: (0, 0)),
            pl.BlockSpec((1, KD), lambda i: (0, 0)),
        ],
        out_specs=[
            pl.BlockSpec((1, NP), lambda i: (0, 0)),
            pl.BlockSpec((1, TOPK), lambda i: (0, 0)),
            pl.BlockSpec((1, TOPK), lambda i: (0, 0)),
        ],
        out_shape=[
            jax.ShapeDtypeStruct((1, NP), jnp.float32),
            jax.ShapeDtypeStruct((1, TOPK), jnp.float32),
            jax.ShapeDtypeStruct((1, TOPK), jnp.int32),
        ],
        scratch_shapes=[pltpu.VMEM((NP, KD), jnp.float32)],
    )(patches, Wq * LOG2E, (bq * LOG2E).reshape(1, QD),
      Wk, bk.reshape(1, KD))
    return bests[0], idx[0], colsum[0]


# speedup vs baseline: 1.1425x; 1.0031x over previous
import jax
import jax.numpy as jnp
from jax.experimental import pallas as pl
from jax.experimental.pallas import tpu as pltpu

H, W, C = 512, 512, 3
S = 8
QD, KD = 32, 32
TOPK = 64
NP = (H // S) * (W // S)
PDIM = S * S * C
RB = 512
NBLK = NP // RB
LOG2E = 1.4426950408889634


def _fused_kernel(patches_ref, wq_ref, bq_ref, wk_ref, bk_ref,
                  colsum_ref, bests_ref, idx_ref, k_scratch):
    i = pl.program_id(0)

    @pl.when(i == 0)
    def _init():
        k_scratch[...] = (
            jnp.dot(patches_ref[...], wk_ref[...],
                    preferred_element_type=jnp.float32) + bk_ref[...])
        colsum_ref[...] = jnp.zeros_like(colsum_ref)

    p_blk = patches_ref[pl.ds(i * RB, RB), :]
    q = (jnp.dot(p_blk, wq_ref[...], preferred_element_type=jnp.float32)
         + bq_ref[...])
    s = jax.lax.dot_general(
        q, k_scratch[...], (((1,), (1,)), ((), ())),
        preferred_element_type=jnp.float32)
    m = jnp.max(s, axis=1, keepdims=True)
    p = jnp.exp(s - m)
    z = jnp.sum(p, axis=1, keepdims=True)
    colsum_ref[...] += jnp.sum(p / z, axis=0, keepdims=True)

    @pl.when(i == NBLK - 1)
    def _topk():
        lanes = jax.lax.broadcasted_iota(jnp.int32, (1, NP), 1)
        tlanes = jax.lax.broadcasted_iota(jnp.int32, (1, TOPK), 1)

        def body(t, carry):
            cur, bvals, bidx = carry
            mval = jnp.max(cur)
            midx = jnp.min(jnp.where(cur == mval, lanes, NP))
            bvals = jnp.where(tlanes == t, mval, bvals)
            bidx = jnp.where(tlanes == t, midx, bidx)
            cur = jnp.where(lanes == midx, -jnp.inf, cur)
            return cur, bvals, bidx

        _, bvals, bidx = jax.lax.fori_loop(
            0, TOPK, body,
            (colsum_ref[...],
             jnp.zeros((1, TOPK), jnp.float32),
             jnp.zeros((1, TOPK), jnp.int32)))
        bests_ref[...] = bvals
        idx_ref[...] = bidx


def kernel(input, Wq, bq, Wk, bk):
    patches = input.reshape(H // S, S, W // S, S * C)
    patches = patches.transpose(0, 2, 1, 3).reshape(NP, PDIM)
    colsum, bests, idx = pl.pallas_call(
        _fused_kernel,
        grid=(NBLK,),
        in_specs=[
            pl.BlockSpec((NP, PDIM), lambda i: (0, 0)),
            pl.BlockSpec((PDIM, QD), lambda i: (0, 0)),
            pl.BlockSpec((1, QD), lambda i: (0, 0)),
            pl.BlockSpec((PDIM, KD), lambda i: (0, 0)),
            pl.BlockSpec((1, KD), lambda i: (0, 0)),
        ],
        out_specs=[
            pl.BlockSpec((1, NP), lambda i: (0, 0)),
            pl.BlockSpec((1, TOPK), lambda i: (0, 0)),
            pl.BlockSpec((1, TOPK), lambda i: (0, 0)),
        ],
        out_shape=[
            jax.ShapeDtypeStruct((1, NP), jnp.float32),
            jax.ShapeDtypeStruct((1, TOPK), jnp.float32),
            jax.ShapeDtypeStruct((1, TOPK), jnp.int32),
        ],
        scratch_shapes=[pltpu.VMEM((NP, KD), jnp.float32)],
    )(patches, Wq, bq.reshape(1, QD), Wk, bk.reshape(1, KD))
    return bests[0], idx[0], colsum[0]


# speedup vs baseline: 1.1998x; 1.0502x over previous
import jax
import jax.numpy as jnp
from jax.experimental import pallas as pl
from jax.experimental.pallas import tpu as pltpu

H, W, C = 512, 512, 3
S = 8
QD, KD = 32, 32
TOPK = 64
NP = (H // S) * (W // S)
PDIM = S * S * C
RB = 512
NBLK = NP // RB
LOG2E = 1.4426950408889634


def _fused_kernel(patches_ref, wq_ref, bq_ref, wk_ref, bk_ref,
                  colsum_ref, bests_ref, idx_ref, k_scratch):
    i = pl.program_id(0)

    @pl.when(i == 0)
    def _init():
        k_scratch[...] = (
            jnp.dot(patches_ref[...], wk_ref[...],
                    preferred_element_type=jnp.float32) + bk_ref[...])
        colsum_ref[...] = jnp.zeros_like(colsum_ref)

    p_blk = patches_ref[pl.ds(i * RB, RB), :]
    q = (jnp.dot(p_blk, wq_ref[...], preferred_element_type=jnp.float32)
         + bq_ref[...])
    s = jax.lax.dot_general(
        q, k_scratch[...], (((1,), (1,)), ((), ())),
        preferred_element_type=jnp.float32)
    m = jnp.max(s, axis=1, keepdims=True)
    p = jnp.exp(s - m)
    z = jnp.sum(p, axis=1, keepdims=True)
    colsum_ref[...] += jnp.sum(p / z, axis=0, keepdims=True)

    @pl.when(i == NBLK - 1)
    def _topk():
        FR = 8
        FC = NP // FR
        cs = colsum_ref[...].reshape(FR, FC)
        gidx = (jax.lax.broadcasted_iota(jnp.int32, (FR, FC), 0) * FC
                + jax.lax.broadcasted_iota(jnp.int32, (FR, FC), 1))
        tlanes = jax.lax.broadcasted_iota(jnp.int32, (1, TOPK), 1)

        def body(t, carry):
            cur, bvals, bidx = carry
            mval = jnp.max(cur)
            midx = jnp.min(jnp.where(cur == mval, gidx, NP))
            bvals = jnp.where(tlanes == t, mval, bvals)
            bidx = jnp.where(tlanes == t, midx, bidx)
            cur = jnp.where(gidx == midx, -jnp.inf, cur)
            return cur, bvals, bidx

        _, bvals, bidx = jax.lax.fori_loop(
            0, TOPK, body,
            (cs,
             jnp.zeros((1, TOPK), jnp.float32),
             jnp.zeros((1, TOPK), jnp.int32)))
        bests_ref[...] = bvals
        idx_ref[...] = bidx


def kernel(input, Wq, bq, Wk, bk):
    patches = input.reshape(H // S, S, W // S, S * C)
    patches = patches.transpose(0, 2, 1, 3).reshape(NP, PDIM)
    colsum, bests, idx = pl.pallas_call(
        _fused_kernel,
        grid=(NBLK,),
        in_specs=[
            pl.BlockSpec((NP, PDIM), lambda i: (0, 0)),
            pl.BlockSpec((PDIM, QD), lambda i: (0, 0)),
            pl.BlockSpec((1, QD), lambda i: (0, 0)),
            pl.BlockSpec((PDIM, KD), lambda i: (0, 0)),
            pl.BlockSpec((1, KD), lambda i: (0, 0)),
        ],
        out_specs=[
            pl.BlockSpec((1, NP), lambda i: (0, 0)),
            pl.BlockSpec((1, TOPK), lambda i: (0, 0)),
            pl.BlockSpec((1, TOPK), lambda i: (0, 0)),
        ],
        out_shape=[
            jax.ShapeDtypeStruct((1, NP), jnp.float32),
            jax.ShapeDtypeStruct((1, TOPK), jnp.float32),
            jax.ShapeDtypeStruct((1, TOPK), jnp.int32),
        ],
        scratch_shapes=[pltpu.VMEM((NP, KD), jnp.float32)],
    )(patches, Wq, bq.reshape(1, QD), Wk, bk.reshape(1, KD))
    return bests[0], idx[0], colsum[0]
